# baseline (speedup 1.0000x reference)
import functools

import jax
import jax.numpy as jnp
from jax import lax
from jax.experimental import pallas as pl
from jax.experimental.pallas import tpu as pltpu
from jax.experimental.pallas import tpu_sc as plsc

N = 10000
E = 320000
NW = 32
NPT = 313
NPAD = NW * NPT
SLOT = 16000
CE = 4000
G = 128

_MESH = plsc.VectorSubcoreMesh(
    core_axis_name="c", subcore_axis_name="s", num_cores=2, num_subcores=16
)


def _wid():
    return lax.axis_index("s") * 2 + lax.axis_index("c")


SLOTH = SLOT // 2
NCH = E // CE


@functools.partial(
    pl.kernel,
    out_type=(
        jax.ShapeDtypeStruct((NW * SLOT,), jnp.int32),
        jax.ShapeDtypeStruct((NW * SLOT,), jnp.int32),
        jax.ShapeDtypeStruct((NW * 16,), jnp.int32),
    ),
    mesh=_MESH,
    compiler_params=pltpu.CompilerParams(needs_layout_passes=False),
    scratch_types=[
        pltpu.VMEM((SLOT,), jnp.int32),
        pltpu.VMEM((SLOT,), jnp.int32),
        pltpu.VMEM((SLOTH,), jnp.int32),
        pltpu.VMEM((SLOTH,), jnp.int32),
        pltpu.VMEM((CE,), jnp.int32),
        pltpu.VMEM((CE,), jnp.int32),
        pltpu.VMEM((CE,), jnp.int32),
        pltpu.VMEM((CE,), jnp.int32),
        pltpu.VMEM((16,), jnp.int32),
        pltpu.SemaphoreType.DMA,
        pltpu.SemaphoreType.DMA,
    ],
)
def _prepass(src_hbm, dst_hbm, bsrc_hbm, bdst_hbm, cnt_hbm,
             bsrcA, bdstA, bsrcB, bdstB,
             src0, src1, dst0, dst1, cnt_v, sem0, sem1):
    w = _wid()
    base = w * NPT
    zeros16 = jnp.zeros((16,), jnp.int32)
    sent16 = jnp.full((16,), NPT * 128, jnp.int32)
    srcs = (src0, src1)
    dsts = (dst0, dst1)
    sems = (sem0, sem1)

    def zero_body(i, _):
        bsrcA[pl.ds(i * 16, 16)] = zeros16
        bdstA[pl.ds(i * 16, 16)] = sent16
        return 0
    lax.fori_loop(0, SLOT // 16, zero_body, 0)

    def zero_body_b(i, _):
        bsrcB[pl.ds(i * 16, 16)] = zeros16
        bdstB[pl.ds(i * 16, 16)] = sent16
        return 0
    lax.fori_loop(0, SLOTH // 16, zero_body_b, 0)

    def start_chunk(c, b):
        pltpu.async_copy(src_hbm.at[pl.ds(c * CE, CE)], srcs[b], sems[b])
        pltpu.async_copy(dst_hbm.at[pl.ds(c * CE, CE)], dsts[b], sems[b])

    def wait_chunk(c, b):
        pltpu.make_async_copy(src_hbm.at[pl.ds(c * CE, CE)], srcs[b],
                              sems[b]).wait()
        pltpu.make_async_copy(dst_hbm.at[pl.ds(c * CE, CE)], dsts[b],
                              sems[b]).wait()

    start_chunk(0, 0)
    start_chunk(1, 1)

    def one_stream(dv, sv, off, bsrc_v, bdst_v):
        loc = dv - base
        mask = plsc.bitcast(loc, jnp.uint32) < jnp.uint32(NPT)
        mi = jnp.where(mask, 1, 0)
        pos = off + jnp.cumsum(mi) - 1
        plsc.store_scatter(bsrc_v, [pos], sv, mask=mask)
        plsc.store_scatter(bdst_v, [pos], loc * 128, mask=mask)
        return pos[15] + 1

    def pair_body(pi, offs):
        for b in range(2):
            c = pi * 2 + b
            wait_chunk(c, b)

            def vec_body(i, offs):
                offA, offB = offs
                iA = i * 32
                iB = iA + 16
                dA = dsts[b][pl.ds(iA, 16)]
                sA = srcs[b][pl.ds(iA, 16)]
                dB = dsts[b][pl.ds(iB, 16)]
                sB = srcs[b][pl.ds(iB, 16)]
                offA = one_stream(dA, sA, offA, bsrcA, bdstA)
                offB = one_stream(dB, sB, offB, bsrcB, bdstB)
                return (offA, offB)
            offs = lax.fori_loop(0, CE // 32, vec_body, offs)

            @pl.when(c + 2 < NCH)
            def _():
                start_chunk(c + 2, b)
        return offs

    offA, offB = lax.fori_loop(0, NCH // 2, pair_body,
                               (jnp.int32(0), jnp.int32(0)))

    rupA = ((offA + 15) // 16) * 16
    nB16 = (offB + 15) // 16

    def merge_body(i, _):
        sv = bsrcB[pl.ds(i * 16, 16)]
        dv = bdstB[pl.ds(i * 16, 16)]
        bsrcA[pl.ds(rupA + i * 16, 16)] = sv
        bdstA[pl.ds(rupA + i * 16, 16)] = dv
        return 0
    lax.fori_loop(0, nB16, merge_body, 0)

    cnt_v[...] = jnp.full((16,), rupA + offB, jnp.int32)
    pltpu.sync_copy(cnt_v, cnt_hbm.at[pl.ds(w * 16, 16)])
    pltpu.sync_copy(bsrcA, bsrc_hbm.at[pl.ds(w * SLOT, SLOT)])
    pltpu.sync_copy(bdstA, bdst_hbm.at[pl.ds(w * SLOT, SLOT)])


def _make_segmax(D):
    assert D == 128
    NW_ACC = NPT * D + D

    @functools.partial(
        pl.kernel,
        out_type=jax.ShapeDtypeStruct((NPAD * D,), jnp.float32),
        mesh=_MESH,
        compiler_params=pltpu.CompilerParams(needs_layout_passes=False),
        scratch_types=[
            pltpu.VMEM((NW_ACC,), jnp.float32),
            pltpu.VMEM((SLOT,), jnp.int32),
            pltpu.VMEM((SLOT,), jnp.int32),
            pltpu.VMEM((G, D), jnp.float32),
            pltpu.VMEM((G, D), jnp.float32),
            pltpu.VMEM((G, D), jnp.float32),
            pltpu.VMEM((16,), jnp.int32),
            pltpu.SemaphoreType.DMA,
            pltpu.SemaphoreType.DMA,
            pltpu.SemaphoreType.DMA,
        ],
    )
    def segmax(p_hbm, bsrc_hbm, bdst_hbm, cnt_hbm, m_hbm,
               acc, idx_all, dl_all, rows0, rows1, rows2, cnt_v,
               sem0, sem1, sem2):
        w = _wid()
        rows = (rows0, rows1, rows2)
        sems = (sem0, sem1, sem2)
        zeros16 = jnp.zeros((16,), jnp.float32)

        def zero_body(i, _):
            acc[pl.ds(i * 16, 16)] = zeros16
            return 0
        lax.fori_loop(0, NW_ACC // 16, zero_body, 0)

        pltpu.sync_copy(cnt_hbm.at[pl.ds(w * 16, 16)], cnt_v)
        ne = cnt_v[pl.ds(0, 16)][0]
        nsb = (ne + (G - 1)) // G
        pltpu.sync_copy(bsrc_hbm.at[pl.ds(w * SLOT, SLOT)], idx_all)
        pltpu.sync_copy(bdst_hbm.at[pl.ds(w * SLOT, SLOT)], dl_all)

        def start(g, b):
            pltpu.async_copy(p_hbm.at[idx_all.at[pl.ds(g * G, G)]], rows[b],
                             sems[b])

        def wait(g, b):
            pltpu.make_async_copy(p_hbm.at[idx_all.at[pl.ds(g * G, G)]],
                                  rows[b], sems[b]).wait()

        for gg in range(3):
            @pl.when(nsb > gg)
            def _(gg=gg):
                start(gg, gg)

        def pair_body(pi, _):
            for b in range(3):
                g = pi * 3 + b

                @pl.when(g < nsb)
                def _():
                    wait(g, b)

                    def jb_body(jb, _):
                        d16 = dl_all[pl.ds(g * G + jb * 16, 16)]
                        nk = D // 16

                        def rloads(t):
                            rb = jb * 16 + t
                            return [rows[b][rb, pl.ds(kk * 16, 16)]
                                    for kk in range(nk)]

                        rv = rloads(0)
                        for t in range(16):
                            ab = d16[t]
                            av = [acc[pl.ds(ab + kk * 16, 16)]
                                  for kk in range(nk)]
                            rv_next = rloads(t + 1) if t < 15 else None
                            for kk in range(nk):
                                acc[pl.ds(ab + kk * 16, 16)] = jnp.maximum(
                                    av[kk], rv[kk])
                            rv = rv_next
                        return 0
                    lax.fori_loop(0, G // 16, jb_body, 0)

                    @pl.when(g + 3 < nsb)
                    def _():
                        start(g + 3, b)
            return 0
        lax.fori_loop(0, (nsb + 2) // 3, pair_body, 0)
        pltpu.sync_copy(acc.at[pl.ds(0, NPT * D)],
                        m_hbm.at[pl.ds(w * (NPT * D), NPT * D)])

    return segmax


_segmax128 = _make_segmax(128)


R = 400
_GRID = (N // R,)


def _row_spec(d):
    return pl.BlockSpec((R, d), lambda i: (i, 0))


def _full_spec(a, b):
    return pl.BlockSpec((a, b), lambda i: (0, 0))


def _stage_a_body(x_ref, wpT, bp, wsT, p_ref, s_ref):
    xb = x_ref[...]
    p_ref[...] = jnp.maximum(
        jnp.dot(xb, wpT[...], preferred_element_type=jnp.float32) + bp[...],
        0.0)
    s_ref[...] = jnp.dot(xb, wsT[...], preferred_element_type=jnp.float32)


def _stage_b_body(s_ref, m_ref, wnT, b, wpT, bp, wsT, p_ref, s2_ref):
    t = s_ref[...] + jnp.dot(m_ref[...], wnT[...],
                             preferred_element_type=jnp.float32) + b[...]
    h = jnp.where(t >= 0, t, 0.01 * t)
    p_ref[...] = jnp.maximum(
        jnp.dot(h, wpT[...], preferred_element_type=jnp.float32) + bp[...],
        0.0)
    s2_ref[...] = jnp.dot(h, wsT[...], preferred_element_type=jnp.float32)


def _stage_d_body(s_ref, m_ref, wnT, b, wlT, bl, o_ref):
    t = s_ref[...] + jnp.dot(m_ref[...], wnT[...],
                             preferred_element_type=jnp.float32) + b[...]
    h = jnp.where(t >= 0, t, 0.01 * t)
    o_ref[...] = jax.nn.sigmoid(
        jnp.dot(h, wlT[...], preferred_element_type=jnp.float32) + bl[...])


def _stage_a(x, wpT, bp, wsT):
    din, dp = wpT.shape
    ds_ = wsT.shape[1]
    return pl.pallas_call(
        _stage_a_body,
        grid=_GRID,
        in_specs=[_row_spec(din), _full_spec(din, dp), _full_spec(1, dp),
                  _full_spec(din, ds_)],
        out_specs=[_row_spec(dp), _row_spec(ds_)],
        out_shape=[jax.ShapeDtypeStruct((N, dp), jnp.float32),
                   jax.ShapeDtypeStruct((N, ds_), jnp.float32)],
    )(x, wpT, bp, wsT)


def _stage_b(s, m, wnT, b, wpT, bp, wsT):
    dm, dh = wnT.shape
    dp = wpT.shape[1]
    ds2 = wsT.shape[1]
    return pl.pallas_call(
        _stage_b_body,
        grid=_GRID,
        in_specs=[_row_spec(dh), _row_spec(dm), _full_spec(dm, dh),
                  _full_spec(1, dh), _full_spec(dh, dp), _full_spec(1, dp),
                  _full_spec(dh, ds2)],
        out_specs=[_row_spec(dp), _row_spec(ds2)],
        out_shape=[jax.ShapeDtypeStruct((N, dp), jnp.float32),
                   jax.ShapeDtypeStruct((N, ds2), jnp.float32)],
    )(s, m, wnT, b, wpT, bp, wsT)


def _stage_d(s, m, wnT, b, wlT, bl):
    dm, dh = wnT.shape
    c = wlT.shape[1]
    return pl.pallas_call(
        _stage_d_body,
        grid=_GRID,
        in_specs=[_row_spec(dh), _row_spec(dm), _full_spec(dm, dh),
                  _full_spec(1, dh), _full_spec(dh, c), _full_spec(1, c)],
        out_specs=_row_spec(c),
        out_shape=jax.ShapeDtypeStruct((N, c), jnp.float32),
    )(s, m, wnT, b, wlT, bl)


def kernel(x, edge_index, Wp1, bp1, Ws1, Wn1, b1, Wp2, bp2, Ws2, Wn2, b2,
           Wp3, bp3, Ws3, Wn3, b3, Wl, bl):
    src = edge_index[0]
    dst = edge_index[1]
    bsrc, bdst, cnt = _prepass(src, dst)

    def seg(p):
        m_flat = _segmax128(p, bsrc, bdst, cnt)
        return m_flat.reshape(NPAD, 128)[:N]

    p1, s1 = _stage_a(x, Wp1.T, bp1.reshape(1, -1), Ws1.T)
    m1 = seg(p1)
    p2, s2 = _stage_b(s1, m1, Wn1.T, b1.reshape(1, -1), Wp2.T,
                      bp2.reshape(1, -1), Ws2.T)
    m2 = seg(p2)
    wp3T_pad = jnp.concatenate([Wp3.T, jnp.zeros((64, 64), jnp.float32)], axis=1)
    bp3_pad = jnp.concatenate([bp3, jnp.zeros((64,), jnp.float32)]).reshape(1, -1)
    p3, s3 = _stage_b(s2, m2, Wn2.T, b2.reshape(1, -1), wp3T_pad,
                      bp3_pad, Ws3.T)
    m3 = seg(p3)
    wn3T_pad = jnp.concatenate([Wn3.T, jnp.zeros((64, 64), jnp.float32)], axis=0)
    return _stage_d(s3, m3, wn3T_pad, b3.reshape(1, -1), Wl.T, bl.reshape(1, -1))


# speedup vs baseline: 1.0340x; 1.0340x over previous
import functools

import jax
import jax.numpy as jnp
from jax import lax
from jax.experimental import pallas as pl
from jax.experimental.pallas import tpu as pltpu
from jax.experimental.pallas import tpu_sc as plsc

N = 10000
E = 320000
NW = 32
NPT = 313
NPAD = NW * NPT
SLOT = 16000
CE = 4000
G = 128

_MESH = plsc.VectorSubcoreMesh(
    core_axis_name="c", subcore_axis_name="s", num_cores=2, num_subcores=16
)


def _wid():
    return lax.axis_index("s") * 2 + lax.axis_index("c")


SLOTH = SLOT // 2
NCH = E // CE


@functools.partial(
    pl.kernel,
    out_type=(
        jax.ShapeDtypeStruct((NW * SLOT,), jnp.int32),
        jax.ShapeDtypeStruct((NW * SLOT,), jnp.int32),
        jax.ShapeDtypeStruct((NW * 16,), jnp.int32),
    ),
    mesh=_MESH,
    compiler_params=pltpu.CompilerParams(needs_layout_passes=False),
    scratch_types=[
        pltpu.VMEM((SLOT,), jnp.int32),
        pltpu.VMEM((SLOT,), jnp.int32),
        pltpu.VMEM((SLOTH,), jnp.int32),
        pltpu.VMEM((SLOTH,), jnp.int32),
        pltpu.VMEM((CE,), jnp.int32),
        pltpu.VMEM((CE,), jnp.int32),
        pltpu.VMEM((CE,), jnp.int32),
        pltpu.VMEM((CE,), jnp.int32),
        pltpu.VMEM((16,), jnp.int32),
        pltpu.SemaphoreType.DMA,
        pltpu.SemaphoreType.DMA,
    ],
)
def _prepass(src_hbm, dst_hbm, bsrc_hbm, bdst_hbm, cnt_hbm,
             bsrcA, bdstA, bsrcB, bdstB,
             src0, src1, dst0, dst1, cnt_v, sem0, sem1):
    w = _wid()
    base = w * NPT
    zeros16 = jnp.zeros((16,), jnp.int32)
    sent16 = jnp.full((16,), NPT * 128, jnp.int32)
    srcs = (src0, src1)
    dsts = (dst0, dst1)
    sems = (sem0, sem1)

    def zero_body(i, _):
        bsrcA[pl.ds(i * 16, 16)] = zeros16
        bdstA[pl.ds(i * 16, 16)] = sent16
        return 0
    lax.fori_loop(0, SLOT // 16, zero_body, 0)

    def zero_body_b(i, _):
        bsrcB[pl.ds(i * 16, 16)] = zeros16
        bdstB[pl.ds(i * 16, 16)] = sent16
        return 0
    lax.fori_loop(0, SLOTH // 16, zero_body_b, 0)

    def start_chunk(c, b):
        pltpu.async_copy(src_hbm.at[pl.ds(c * CE, CE)], srcs[b], sems[b])
        pltpu.async_copy(dst_hbm.at[pl.ds(c * CE, CE)], dsts[b], sems[b])

    def wait_chunk(c, b):
        pltpu.make_async_copy(src_hbm.at[pl.ds(c * CE, CE)], srcs[b],
                              sems[b]).wait()
        pltpu.make_async_copy(dst_hbm.at[pl.ds(c * CE, CE)], dsts[b],
                              sems[b]).wait()

    start_chunk(0, 0)
    start_chunk(1, 1)

    def one_stream(dv, sv, off, bsrc_v, bdst_v):
        loc = dv - base
        mask = plsc.bitcast(loc, jnp.uint32) < jnp.uint32(NPT)
        mi = jnp.where(mask, 1, 0)
        pos = off + jnp.cumsum(mi) - 1
        plsc.store_scatter(bsrc_v, [pos], sv, mask=mask)
        plsc.store_scatter(bdst_v, [pos], loc * 128, mask=mask)
        return pos[15] + 1

    def pair_body(pi, offs):
        for b in range(2):
            c = pi * 2 + b
            wait_chunk(c, b)

            def vec_body(i, offs):
                offA, offB = offs
                iA = i * 32
                iB = iA + 16
                dA = dsts[b][pl.ds(iA, 16)]
                sA = srcs[b][pl.ds(iA, 16)]
                dB = dsts[b][pl.ds(iB, 16)]
                sB = srcs[b][pl.ds(iB, 16)]
                offA = one_stream(dA, sA, offA, bsrcA, bdstA)
                offB = one_stream(dB, sB, offB, bsrcB, bdstB)
                return (offA, offB)
            offs = lax.fori_loop(0, CE // 32, vec_body, offs)

            @pl.when(c + 2 < NCH)
            def _():
                start_chunk(c + 2, b)
        return offs

    offA, offB = lax.fori_loop(0, NCH // 2, pair_body,
                               (jnp.int32(0), jnp.int32(0)))

    rupA = ((offA + 15) // 16) * 16
    nB16 = (offB + 15) // 16

    def merge_body(i, _):
        sv = bsrcB[pl.ds(i * 16, 16)]
        dv = bdstB[pl.ds(i * 16, 16)]
        bsrcA[pl.ds(rupA + i * 16, 16)] = sv
        bdstA[pl.ds(rupA + i * 16, 16)] = dv
        return 0
    lax.fori_loop(0, nB16, merge_body, 0)

    cnt_v[...] = jnp.full((16,), rupA + offB, jnp.int32)
    pltpu.sync_copy(cnt_v, cnt_hbm.at[pl.ds(w * 16, 16)])
    pltpu.sync_copy(bsrcA, bsrc_hbm.at[pl.ds(w * SLOT, SLOT)])
    pltpu.sync_copy(bdstA, bdst_hbm.at[pl.ds(w * SLOT, SLOT)])


def _make_segmax(D):
    assert D == 128
    NW_ACC = NPT * D + D

    @functools.partial(
        pl.kernel,
        out_type=jax.ShapeDtypeStruct((NPAD * D,), jnp.float32),
        mesh=_MESH,
        compiler_params=pltpu.CompilerParams(needs_layout_passes=False),
        scratch_types=[
            pltpu.VMEM((NW_ACC,), jnp.float32),
            pltpu.VMEM((SLOT,), jnp.int32),
            pltpu.VMEM((SLOT,), jnp.int32),
            pltpu.VMEM((G, D), jnp.float32),
            pltpu.VMEM((G, D), jnp.float32),
            pltpu.VMEM((G, D), jnp.float32),
            pltpu.VMEM((16,), jnp.int32),
            pltpu.SemaphoreType.DMA,
            pltpu.SemaphoreType.DMA,
            pltpu.SemaphoreType.DMA,
        ],
    )
    def segmax(p_hbm, bsrc_hbm, bdst_hbm, cnt_hbm, m_hbm,
               acc, idx_all, dl_all, rows0, rows1, rows2, cnt_v,
               sem0, sem1, sem2):
        w = _wid()
        rows = (rows0, rows1, rows2)
        sems = (sem0, sem1, sem2)
        zeros16 = jnp.zeros((16,), jnp.float32)

        def zero_body(i, _):
            acc[pl.ds(i * 16, 16)] = zeros16
            return 0
        lax.fori_loop(0, NW_ACC // 16, zero_body, 0)

        pltpu.sync_copy(cnt_hbm.at[pl.ds(w * 16, 16)], cnt_v)
        ne = cnt_v[pl.ds(0, 16)][0]
        nsb = (ne + (G - 1)) // G
        pltpu.sync_copy(bsrc_hbm.at[pl.ds(w * SLOT, SLOT)], idx_all)
        pltpu.sync_copy(bdst_hbm.at[pl.ds(w * SLOT, SLOT)], dl_all)

        def start(g, b):
            pltpu.async_copy(p_hbm.at[idx_all.at[pl.ds(g * G, G)]], rows[b],
                             sems[b])

        def wait(g, b):
            pltpu.make_async_copy(p_hbm.at[idx_all.at[pl.ds(g * G, G)]],
                                  rows[b], sems[b]).wait()

        for gg in range(3):
            @pl.when(nsb > gg)
            def _(gg=gg):
                start(gg, gg)

        def pair_body(pi, _):
            for b in range(3):
                g = pi * 3 + b

                @pl.when(g < nsb)
                def _():
                    wait(g, b)

                    def jb_body(jb, _):
                        d16 = dl_all[pl.ds(g * G + jb * 16, 16)]
                        nk = D // 16

                        def rloads(t):
                            rb = jb * 16 + t
                            return [rows[b][rb, pl.ds(kk * 16, 16)]
                                    for kk in range(nk)]

                        rv = rloads(0)
                        for t in range(16):
                            ab = d16[t]
                            av = [acc[pl.ds(ab + kk * 16, 16)]
                                  for kk in range(nk)]
                            rv_next = rloads(t + 1) if t < 15 else None
                            for kk in range(nk):
                                acc[pl.ds(ab + kk * 16, 16)] = jnp.maximum(
                                    av[kk], rv[kk])
                            rv = rv_next
                        return 0
                    lax.fori_loop(0, G // 16, jb_body, 0)

                    @pl.when(g + 3 < nsb)
                    def _():
                        start(g + 3, b)
            return 0
        lax.fori_loop(0, (nsb + 2) // 3, pair_body, 0)
        pltpu.sync_copy(acc.at[pl.ds(0, NPT * D)],
                        m_hbm.at[pl.ds(w * (NPT * D), NPT * D)])

    return segmax


_segmax128 = _make_segmax(128)


R = 2000
_GRID = (N // R,)


def _row_spec(d):
    return pl.BlockSpec((R, d), lambda i: (i, 0))


def _full_spec(a, b):
    return pl.BlockSpec((a, b), lambda i: (0, 0))


def _stage_a_body(x_ref, wpT, bp, wsT, p_ref, s_ref):
    xb = x_ref[...]
    p_ref[...] = jnp.maximum(
        jnp.dot(xb, wpT[...], preferred_element_type=jnp.float32) + bp[...],
        0.0)
    s_ref[...] = jnp.dot(xb, wsT[...], preferred_element_type=jnp.float32)


def _stage_b_body(s_ref, m_ref, wnT, b, wpT, bp, wsT, p_ref, s2_ref):
    t = s_ref[...] + jnp.dot(m_ref[...], wnT[...],
                             preferred_element_type=jnp.float32) + b[...]
    h = jnp.where(t >= 0, t, 0.01 * t)
    p_ref[...] = jnp.maximum(
        jnp.dot(h, wpT[...], preferred_element_type=jnp.float32) + bp[...],
        0.0)
    s2_ref[...] = jnp.dot(h, wsT[...], preferred_element_type=jnp.float32)


def _stage_d_body(s_ref, m_ref, wnT, b, wlT, bl, o_ref):
    t = s_ref[...] + jnp.dot(m_ref[...], wnT[...],
                             preferred_element_type=jnp.float32) + b[...]
    h = jnp.where(t >= 0, t, 0.01 * t)
    o_ref[...] = jax.nn.sigmoid(
        jnp.dot(h, wlT[...], preferred_element_type=jnp.float32) + bl[...])


def _stage_a(x, wpT, bp, wsT):
    din, dp = wpT.shape
    ds_ = wsT.shape[1]
    return pl.pallas_call(
        _stage_a_body,
        grid=_GRID,
        in_specs=[_row_spec(din), _full_spec(din, dp), _full_spec(1, dp),
                  _full_spec(din, ds_)],
        out_specs=[_row_spec(dp), _row_spec(ds_)],
        out_shape=[jax.ShapeDtypeStruct((N, dp), jnp.float32),
                   jax.ShapeDtypeStruct((N, ds_), jnp.float32)],
    )(x, wpT, bp, wsT)


def _stage_b(s, m, wnT, b, wpT, bp, wsT):
    dm, dh = wnT.shape
    dp = wpT.shape[1]
    ds2 = wsT.shape[1]
    return pl.pallas_call(
        _stage_b_body,
        grid=_GRID,
        in_specs=[_row_spec(dh), _row_spec(dm), _full_spec(dm, dh),
                  _full_spec(1, dh), _full_spec(dh, dp), _full_spec(1, dp),
                  _full_spec(dh, ds2)],
        out_specs=[_row_spec(dp), _row_spec(ds2)],
        out_shape=[jax.ShapeDtypeStruct((N, dp), jnp.float32),
                   jax.ShapeDtypeStruct((N, ds2), jnp.float32)],
    )(s, m, wnT, b, wpT, bp, wsT)


def _stage_d(s, m, wnT, b, wlT, bl):
    dm, dh = wnT.shape
    c = wlT.shape[1]
    return pl.pallas_call(
        _stage_d_body,
        grid=_GRID,
        in_specs=[_row_spec(dh), _row_spec(dm), _full_spec(dm, dh),
                  _full_spec(1, dh), _full_spec(dh, c), _full_spec(1, c)],
        out_specs=_row_spec(c),
        out_shape=jax.ShapeDtypeStruct((N, c), jnp.float32),
    )(s, m, wnT, b, wlT, bl)


def kernel(x, edge_index, Wp1, bp1, Ws1, Wn1, b1, Wp2, bp2, Ws2, Wn2, b2,
           Wp3, bp3, Ws3, Wn3, b3, Wl, bl):
    src = edge_index[0]
    dst = edge_index[1]
    bsrc, bdst, cnt = _prepass(src, dst)

    def seg(p):
        m_flat = _segmax128(p, bsrc, bdst, cnt)
        return m_flat.reshape(NPAD, 128)[:N]

    p1, s1 = _stage_a(x, Wp1.T, bp1.reshape(1, -1), Ws1.T)
    m1 = seg(p1)
    p2, s2 = _stage_b(s1, m1, Wn1.T, b1.reshape(1, -1), Wp2.T,
                      bp2.reshape(1, -1), Ws2.T)
    m2 = seg(p2)
    wp3T_pad = jnp.concatenate([Wp3.T, jnp.zeros((64, 64), jnp.float32)], axis=1)
    bp3_pad = jnp.concatenate([bp3, jnp.zeros((64,), jnp.float32)]).reshape(1, -1)
    p3, s3 = _stage_b(s2, m2, Wn2.T, b2.reshape(1, -1), wp3T_pad,
                      bp3_pad, Ws3.T)
    m3 = seg(p3)
    wn3T_pad = jnp.concatenate([Wn3.T, jnp.zeros((64, 64), jnp.float32)], axis=0)
    return _stage_d(s3, m3, wn3T_pad, b3.reshape(1, -1), Wl.T, bl.reshape(1, -1))


# speedup vs baseline: 1.0349x; 1.0009x over previous
import functools

import jax
import jax.numpy as jnp
from jax import lax
from jax.experimental import pallas as pl
from jax.experimental.pallas import tpu as pltpu
from jax.experimental.pallas import tpu_sc as plsc

N = 10000
E = 320000
NW = 32
NPT = 313
NPAD = NW * NPT
SLOT = 16000
CE = 4000
G = 128

_MESH = plsc.VectorSubcoreMesh(
    core_axis_name="c", subcore_axis_name="s", num_cores=2, num_subcores=16
)


def _wid():
    return lax.axis_index("s") * 2 + lax.axis_index("c")


SLOTH = SLOT // 2
NCH = E // CE


@functools.partial(
    pl.kernel,
    out_type=(
        jax.ShapeDtypeStruct((NW * SLOT,), jnp.int32),
        jax.ShapeDtypeStruct((NW * SLOT,), jnp.int32),
        jax.ShapeDtypeStruct((NW * 16,), jnp.int32),
    ),
    mesh=_MESH,
    compiler_params=pltpu.CompilerParams(needs_layout_passes=False),
    scratch_types=[
        pltpu.VMEM((SLOT,), jnp.int32),
        pltpu.VMEM((SLOT,), jnp.int32),
        pltpu.VMEM((SLOTH,), jnp.int32),
        pltpu.VMEM((SLOTH,), jnp.int32),
        pltpu.VMEM((CE,), jnp.int32),
        pltpu.VMEM((CE,), jnp.int32),
        pltpu.VMEM((CE,), jnp.int32),
        pltpu.VMEM((CE,), jnp.int32),
        pltpu.VMEM((16,), jnp.int32),
        pltpu.SemaphoreType.DMA,
        pltpu.SemaphoreType.DMA,
    ],
)
def _prepass(src_hbm, dst_hbm, bsrc_hbm, bdst_hbm, cnt_hbm,
             bsrcA, bdstA, bsrcB, bdstB,
             src0, src1, dst0, dst1, cnt_v, sem0, sem1):
    w = _wid()
    base = w * NPT
    zeros16 = jnp.zeros((16,), jnp.int32)
    sent16 = jnp.full((16,), NPT * 128, jnp.int32)
    srcs = (src0, src1)
    dsts = (dst0, dst1)
    sems = (sem0, sem1)

    def zero_body(i, _):
        bsrcA[pl.ds(i * 16, 16)] = zeros16
        bdstA[pl.ds(i * 16, 16)] = sent16
        return 0
    lax.fori_loop(0, SLOT // 16, zero_body, 0)

    def zero_body_b(i, _):
        bsrcB[pl.ds(i * 16, 16)] = zeros16
        bdstB[pl.ds(i * 16, 16)] = sent16
        return 0
    lax.fori_loop(0, SLOTH // 16, zero_body_b, 0)

    def start_chunk(c, b):
        pltpu.async_copy(src_hbm.at[pl.ds(c * CE, CE)], srcs[b], sems[b])
        pltpu.async_copy(dst_hbm.at[pl.ds(c * CE, CE)], dsts[b], sems[b])

    def wait_chunk(c, b):
        pltpu.make_async_copy(src_hbm.at[pl.ds(c * CE, CE)], srcs[b],
                              sems[b]).wait()
        pltpu.make_async_copy(dst_hbm.at[pl.ds(c * CE, CE)], dsts[b],
                              sems[b]).wait()

    start_chunk(0, 0)
    start_chunk(1, 1)

    def one_stream(dv, sv, off, bsrc_v, bdst_v):
        loc = dv - base
        mask = plsc.bitcast(loc, jnp.uint32) < jnp.uint32(NPT)
        mi = jnp.where(mask, 1, 0)
        pos = off + jnp.cumsum(mi) - 1
        plsc.store_scatter(bsrc_v, [pos], sv, mask=mask)
        plsc.store_scatter(bdst_v, [pos], loc * 128, mask=mask)
        return pos[15] + 1

    def pair_body(pi, offs):
        for b in range(2):
            c = pi * 2 + b
            wait_chunk(c, b)

            def vec_body(i, offs):
                offA, offB = offs
                iA = i * 32
                iB = iA + 16
                dA = dsts[b][pl.ds(iA, 16)]
                sA = srcs[b][pl.ds(iA, 16)]
                dB = dsts[b][pl.ds(iB, 16)]
                sB = srcs[b][pl.ds(iB, 16)]
                offA = one_stream(dA, sA, offA, bsrcA, bdstA)
                offB = one_stream(dB, sB, offB, bsrcB, bdstB)
                return (offA, offB)
            offs = lax.fori_loop(0, CE // 32, vec_body, offs)

            @pl.when(c + 2 < NCH)
            def _():
                start_chunk(c + 2, b)
        return offs

    offA, offB = lax.fori_loop(0, NCH // 2, pair_body,
                               (jnp.int32(0), jnp.int32(0)))

    rupA = ((offA + 15) // 16) * 16
    nB16 = (offB + 15) // 16

    def merge_body(i, _):
        sv = bsrcB[pl.ds(i * 16, 16)]
        dv = bdstB[pl.ds(i * 16, 16)]
        bsrcA[pl.ds(rupA + i * 16, 16)] = sv
        bdstA[pl.ds(rupA + i * 16, 16)] = dv
        return 0
    lax.fori_loop(0, nB16, merge_body, 0)

    cnt_v[...] = jnp.full((16,), rupA + offB, jnp.int32)
    pltpu.sync_copy(cnt_v, cnt_hbm.at[pl.ds(w * 16, 16)])
    pltpu.sync_copy(bsrcA, bsrc_hbm.at[pl.ds(w * SLOT, SLOT)])
    pltpu.sync_copy(bdstA, bdst_hbm.at[pl.ds(w * SLOT, SLOT)])


def _make_segmax(D):
    assert D == 128
    NW_ACC = NPT * D + D

    @functools.partial(
        pl.kernel,
        out_type=jax.ShapeDtypeStruct((NPAD * D,), jnp.float32),
        mesh=_MESH,
        compiler_params=pltpu.CompilerParams(needs_layout_passes=False),
        scratch_types=[
            pltpu.VMEM((NW_ACC,), jnp.float32),
            pltpu.VMEM((SLOT,), jnp.int32),
            pltpu.VMEM((SLOT,), jnp.int32),
            pltpu.VMEM((G, D), jnp.float32),
            pltpu.VMEM((G, D), jnp.float32),
            pltpu.VMEM((G, D), jnp.float32),
            pltpu.VMEM((16,), jnp.int32),
            pltpu.SemaphoreType.DMA,
            pltpu.SemaphoreType.DMA,
            pltpu.SemaphoreType.DMA,
        ],
    )
    def segmax(p_hbm, bsrc_hbm, bdst_hbm, cnt_hbm, m_hbm,
               acc, idx_all, dl_all, rows0, rows1, rows2, cnt_v,
               sem0, sem1, sem2):
        w = _wid()
        rows = (rows0, rows1, rows2)
        sems = (sem0, sem1, sem2)
        zeros16 = jnp.zeros((16,), jnp.float32)

        def zero_body(i, _):
            acc[pl.ds(i * 16, 16)] = zeros16
            return 0
        lax.fori_loop(0, NW_ACC // 16, zero_body, 0)

        pltpu.sync_copy(cnt_hbm.at[pl.ds(w * 16, 16)], cnt_v)
        ne = cnt_v[pl.ds(0, 16)][0]
        nsb = (ne + (G - 1)) // G
        pltpu.sync_copy(bsrc_hbm.at[pl.ds(w * SLOT, SLOT)], idx_all)
        pltpu.sync_copy(bdst_hbm.at[pl.ds(w * SLOT, SLOT)], dl_all)

        def start(g, b):
            pltpu.async_copy(p_hbm.at[idx_all.at[pl.ds(g * G, G)]], rows[b],
                             sems[b])

        def wait(g, b):
            pltpu.make_async_copy(p_hbm.at[idx_all.at[pl.ds(g * G, G)]],
                                  rows[b], sems[b]).wait()

        for gg in range(3):
            @pl.when(nsb > gg)
            def _(gg=gg):
                start(gg, gg)

        def pair_body(pi, _):
            for b in range(3):
                g = pi * 3 + b

                @pl.when(g < nsb)
                def _():
                    wait(g, b)

                    def jb_body(jb, _):
                        d16 = dl_all[pl.ds(g * G + jb * 16, 16)]
                        nk = D // 16

                        def rloads(t):
                            rb = jb * 16 + t
                            return [rows[b][rb, pl.ds(kk * 16, 16)]
                                    for kk in range(nk)]

                        rv = rloads(0)
                        for t in range(16):
                            ab = d16[t]
                            av = [acc[pl.ds(ab + kk * 16, 16)]
                                  for kk in range(nk)]
                            rv_next = rloads(t + 1) if t < 15 else None
                            for kk in range(nk):
                                acc[pl.ds(ab + kk * 16, 16)] = jnp.maximum(
                                    av[kk], rv[kk])
                            rv = rv_next
                        return 0
                    lax.fori_loop(0, G // 16, jb_body, 0)

                    @pl.when(g + 3 < nsb)
                    def _():
                        start(g + 3, b)
            return 0
        lax.fori_loop(0, (nsb + 2) // 3, pair_body, 0)
        pltpu.sync_copy(acc.at[pl.ds(0, NPT * D)],
                        m_hbm.at[pl.ds(w * (NPT * D), NPT * D)])

    return segmax


_segmax128 = _make_segmax(128)


R = 10000
_GRID = (N // R,)


def _row_spec(d):
    return pl.BlockSpec((R, d), lambda i: (i, 0))


def _full_spec(a, b):
    return pl.BlockSpec((a, b), lambda i: (0, 0))


def _stage_a_body(x_ref, wpT, bp, wsT, p_ref, s_ref):
    xb = x_ref[...]
    p_ref[...] = jnp.maximum(
        jnp.dot(xb, wpT[...], preferred_element_type=jnp.float32) + bp[...],
        0.0)
    s_ref[...] = jnp.dot(xb, wsT[...], preferred_element_type=jnp.float32)


def _stage_b_body(s_ref, m_ref, wnT, b, wpT, bp, wsT, p_ref, s2_ref):
    t = s_ref[...] + jnp.dot(m_ref[...], wnT[...],
                             preferred_element_type=jnp.float32) + b[...]
    h = jnp.where(t >= 0, t, 0.01 * t)
    p_ref[...] = jnp.maximum(
        jnp.dot(h, wpT[...], preferred_element_type=jnp.float32) + bp[...],
        0.0)
    s2_ref[...] = jnp.dot(h, wsT[...], preferred_element_type=jnp.float32)


def _stage_d_body(s_ref, m_ref, wnT, b, wlT, bl, o_ref):
    t = s_ref[...] + jnp.dot(m_ref[...], wnT[...],
                             preferred_element_type=jnp.float32) + b[...]
    h = jnp.where(t >= 0, t, 0.01 * t)
    o_ref[...] = jax.nn.sigmoid(
        jnp.dot(h, wlT[...], preferred_element_type=jnp.float32) + bl[...])


def _stage_a(x, wpT, bp, wsT):
    din, dp = wpT.shape
    ds_ = wsT.shape[1]
    return pl.pallas_call(
        _stage_a_body,
        grid=_GRID,
        in_specs=[_row_spec(din), _full_spec(din, dp), _full_spec(1, dp),
                  _full_spec(din, ds_)],
        out_specs=[_row_spec(dp), _row_spec(ds_)],
        out_shape=[jax.ShapeDtypeStruct((N, dp), jnp.float32),
                   jax.ShapeDtypeStruct((N, ds_), jnp.float32)],
    )(x, wpT, bp, wsT)


def _stage_b(s, m, wnT, b, wpT, bp, wsT):
    dm, dh = wnT.shape
    dp = wpT.shape[1]
    ds2 = wsT.shape[1]
    return pl.pallas_call(
        _stage_b_body,
        grid=_GRID,
        in_specs=[_row_spec(dh), _row_spec(dm), _full_spec(dm, dh),
                  _full_spec(1, dh), _full_spec(dh, dp), _full_spec(1, dp),
                  _full_spec(dh, ds2)],
        out_specs=[_row_spec(dp), _row_spec(ds2)],
        out_shape=[jax.ShapeDtypeStruct((N, dp), jnp.float32),
                   jax.ShapeDtypeStruct((N, ds2), jnp.float32)],
    )(s, m, wnT, b, wpT, bp, wsT)


def _stage_d(s, m, wnT, b, wlT, bl):
    dm, dh = wnT.shape
    c = wlT.shape[1]
    return pl.pallas_call(
        _stage_d_body,
        grid=_GRID,
        in_specs=[_row_spec(dh), _row_spec(dm), _full_spec(dm, dh),
                  _full_spec(1, dh), _full_spec(dh, c), _full_spec(1, c)],
        out_specs=_row_spec(c),
        out_shape=jax.ShapeDtypeStruct((N, c), jnp.float32),
    )(s, m, wnT, b, wlT, bl)


def kernel(x, edge_index, Wp1, bp1, Ws1, Wn1, b1, Wp2, bp2, Ws2, Wn2, b2,
           Wp3, bp3, Ws3, Wn3, b3, Wl, bl):
    src = edge_index[0]
    dst = edge_index[1]
    bsrc, bdst, cnt = _prepass(src, dst)

    def seg(p):
        m_flat = _segmax128(p, bsrc, bdst, cnt)
        return m_flat.reshape(NPAD, 128)[:N]

    p1, s1 = _stage_a(x, Wp1.T, bp1.reshape(1, -1), Ws1.T)
    m1 = seg(p1)
    p2, s2 = _stage_b(s1, m1, Wn1.T, b1.reshape(1, -1), Wp2.T,
                      bp2.reshape(1, -1), Ws2.T)
    m2 = seg(p2)
    wp3T_pad = jnp.concatenate([Wp3.T, jnp.zeros((64, 64), jnp.float32)], axis=1)
    bp3_pad = jnp.concatenate([bp3, jnp.zeros((64,), jnp.float32)]).reshape(1, -1)
    p3, s3 = _stage_b(s2, m2, Wn2.T, b2.reshape(1, -1), wp3T_pad,
                      bp3_pad, Ws3.T)
    m3 = seg(p3)
    wn3T_pad = jnp.concatenate([Wn3.T, jnp.zeros((64, 64), jnp.float32)], axis=0)
    return _stage_d(s3, m3, wn3T_pad, b3.reshape(1, -1), Wl.T, bl.reshape(1, -1))
